# baseline (speedup 1.0000x reference)
import functools
import math

import jax
import jax.numpy as jnp
from jax import lax
from jax.experimental import pallas as pl
from jax.experimental.pallas import tpu as pltpu

B = 4096
LAT = 64
OBS = 72
HID = 1024
J = 29
NCMD = 16
K = 8
TOPK = 2
GH = 256
POS = 75
EH = 512
LOG_STD_MIN = math.log(1e-4)
LOG_STD_MAX = math.log(5.0)

TD = 256
S = TOPK * B + K * TD
NT = S // TD
YP = 32

_F32 = jnp.float32


def _elu(x):
    return jnp.where(x > 0, x, jnp.exp(jnp.minimum(x, 0.0)) - 1.0)


def _softmax(x):
    m = jnp.max(x, axis=-1, keepdims=True)
    e = jnp.exp(x - m)
    return e / jnp.sum(e, axis=-1, keepdims=True)


def _ln(x, g, b):
    m = x.mean(-1, keepdims=True)
    v = ((x - m) ** 2).mean(-1, keepdims=True)
    return (x - m) * jax.lax.rsqrt(v + 1e-5) * g + b


def _gate_body(obs_ref, z_ref, chW1, chb1, chW2, chb2, gW1, gb1, gW2, gb2,
               gW3, gb3, lows, highs,
               xrow_out, wn_out, msk_out, rank_out, cnt_out, cnt_acc):
    i = pl.program_id(0)

    @pl.when(i == 0)
    def _():
        cnt_acc[...] = jnp.zeros_like(cnt_acc)

    obs = obs_ref[...]
    z = z_ref[...]
    oz = jnp.concatenate([obs, z], axis=-1)
    h = _elu(jnp.dot(oz, chW1[...], preferred_element_type=_F32) + chb1[...])
    cmd01 = jax.nn.sigmoid(jnp.dot(h, chW2[...], preferred_element_type=_F32) + chb2[...])
    lo = lows[...]
    cmd = lo + (highs[...] - lo) * cmd01
    g_in = jnp.concatenate([cmd, obs[:, NCMD:], z], axis=-1)
    g = _elu(jnp.dot(g_in, gW1[...], preferred_element_type=_F32) + gb1[...])
    g = _elu(jnp.dot(g, gW2[...], preferred_element_type=_F32) + gb2[...])
    logits = jnp.dot(g, gW3[...], preferred_element_type=_F32) + gb3[...]
    w = _softmax(logits)
    kidx = lax.broadcasted_iota(jnp.int32, w.shape, 1)
    m1 = jnp.max(w, axis=-1, keepdims=True)
    i1 = jnp.min(jnp.where(w == m1, kidx, K), axis=-1, keepdims=True)
    w2 = jnp.where(kidx == i1, -1.0, w)
    m2 = jnp.max(w2, axis=-1, keepdims=True)
    i2 = jnp.min(jnp.where(w2 == m2, kidx, K), axis=-1, keepdims=True)
    msk = (kidx == i1) | (kidx == i2)
    mskf = msk.astype(_F32)
    wm = jnp.where(msk, w, 0.0)
    wn = wm / jnp.sum(wm, axis=-1, keepdims=True)
    t = mskf.shape[0]
    r_i = lax.broadcasted_iota(jnp.int32, (t, t), 0)
    c_i = lax.broadcasted_iota(jnp.int32, (t, t), 1)
    tri = (r_i > c_i).astype(_F32)
    r_local = jnp.dot(tri, mskf, preferred_element_type=_F32)
    rank_out[...] = cnt_acc[...] + r_local
    cnt_acc[...] += jnp.sum(mskf, axis=0, keepdims=True)
    cnt_out[...] = cnt_acc[...]
    xrow_out[...] = g_in[:, :OBS]
    wn_out[...] = wn
    msk_out[...] = mskf


def _expert_body(eids_ref, xd_ref, ws_ref, W1f, b1f, W2, b2, W3, b3, y_ref):
    x = xd_ref[...]
    h = _elu(jnp.dot(x, W1f[0], preferred_element_type=_F32) + b1f[0])
    h = _elu(jnp.dot(h, W2[0], preferred_element_type=_F32) + b2[0])
    mu = jnp.dot(h, W3[0], preferred_element_type=_F32) + b3[0]
    wmu = ws_ref[...] * mu
    y_ref[...] = jnp.concatenate(
        [wmu, jnp.zeros((wmu.shape[0], YP - J), _F32)], axis=-1)


def _fn_body(obs_ref, z_ref, mask_ref, fnW1, fnb1, g1, be1, fnW2, fnb2, g2,
             be2, lsW1, lsb1, lsW2, lsb2, vhW, vhb,
             feats_out, ls_out, sig_out):
    oz = jnp.concatenate([obs_ref[...], z_ref[...]], axis=-1)
    x = jnp.dot(oz, fnW1[...], preferred_element_type=_F32) + fnb1[...]
    x = jax.nn.relu(_ln(x, g1[...], be1[...]))
    x = jnp.dot(x, fnW2[...], preferred_element_type=_F32) + fnb2[...]
    x = jax.nn.relu(_ln(x, g2[...], be2[...]))
    feats = x * mask_ref[...]
    h = jax.nn.relu(jnp.dot(feats, lsW1[...], preferred_element_type=_F32) + lsb1[...])
    ls = jnp.dot(h, lsW2[...], preferred_element_type=_F32) + lsb2[...]
    log_std = jnp.clip(ls, LOG_STD_MIN, LOG_STD_MAX)
    sr = jnp.dot(feats, vhW[...], preferred_element_type=_F32) + vhb[...]
    sigma = 0.05 + (0.5 - 0.05) * jax.nn.sigmoid(sr)
    feats_out[...] = feats
    ls_out[...] = log_std
    sig_out[...] = jnp.log(sigma)


def _row_spec(t, n):
    return pl.BlockSpec((t, n), lambda i: (0, 0) if t is None else (i, 0))


def _full_spec(shape):
    nd = len(shape)
    return pl.BlockSpec(shape, lambda i, _nd=nd: (0,) * _nd)


def kernel(z, obs_t, mask_t, params, consts):
    p, c = params, consts
    r2 = lambda a: a.reshape(1, -1)

    TG = 512
    xrow, wn, mskf, rank, cnt = pl.pallas_call(
        _gate_body,
        grid=(B // TG,),
        in_specs=[
            pl.BlockSpec((TG, OBS), lambda i: (i, 0)),
            pl.BlockSpec((TG, LAT), lambda i: (i, 0)),
            _full_spec((OBS + LAT, GH)), _full_spec((1, GH)),
            _full_spec((GH, NCMD)), _full_spec((1, NCMD)),
            _full_spec((OBS + LAT, GH)), _full_spec((1, GH)),
            _full_spec((GH, GH)), _full_spec((1, GH)),
            _full_spec((GH, K)), _full_spec((1, K)),
            _full_spec((1, NCMD)), _full_spec((1, NCMD)),
        ],
        out_specs=[
            pl.BlockSpec((TG, OBS), lambda i: (i, 0)),
            pl.BlockSpec((TG, K), lambda i: (i, 0)),
            pl.BlockSpec((TG, K), lambda i: (i, 0)),
            pl.BlockSpec((TG, K), lambda i: (i, 0)),
            pl.BlockSpec((1, K), lambda i: (0, 0)),
        ],
        out_shape=[
            jax.ShapeDtypeStruct((B, OBS), _F32),
            jax.ShapeDtypeStruct((B, K), _F32),
            jax.ShapeDtypeStruct((B, K), _F32),
            jax.ShapeDtypeStruct((B, K), _F32),
            jax.ShapeDtypeStruct((1, K), _F32),
        ],
        scratch_shapes=[pltpu.VMEM((1, K), _F32)],
        compiler_params=pltpu.CompilerParams(
            dimension_semantics=("arbitrary",)),
    )(obs_t, z, p['ch_W1'], r2(p['ch_b1']), p['ch_W2'], r2(p['ch_b2']),
      p['g_W1'], r2(p['g_b1']), p['g_W2'], r2(p['g_b2']),
      p['g_W3'], r2(p['g_b3']), r2(c['cmd_lows']), r2(c['cmd_highs']))

    n_k = cnt[0].astype(jnp.int32)
    pad_k = ((n_k + TD - 1) // TD) * TD
    cum_pad = jnp.cumsum(pad_k)
    start = cum_pad - pad_k
    slot_all = start[None, :] + rank.astype(jnp.int32)
    mskb = mskf > 0.5
    slot_m = jnp.where(mskb, slot_all, S)
    slot1 = jnp.min(slot_m, axis=-1)
    slot2 = jnp.max(jnp.where(mskb, slot_all, -1), axis=-1)
    tok_ids = jnp.broadcast_to(jnp.arange(B, dtype=jnp.int32)[:, None], (B, K))
    sched_tok = jnp.zeros((S + 1,), jnp.int32).at[slot_m.reshape(-1)].set(
        tok_ids.reshape(-1), mode='drop')[:S]
    w_slot = jnp.zeros((S + 1,), _F32).at[slot_m.reshape(-1)].set(
        wn.reshape(-1), mode='drop')[:S]
    tile_base = jnp.arange(NT, dtype=jnp.int32) * TD
    eids = jnp.minimum(
        jnp.sum((tile_base[:, None] >= cum_pad[None, :]).astype(jnp.int32),
                axis=-1), K - 1)

    x_d = jnp.take(xrow, sched_tok, axis=0)

    W1f = (c['ex_W1'] * jnp.concatenate(
        [c['cmd_masks'][:, :, None],
         jnp.ones((K, OBS - NCMD, 1), _F32)], axis=1))
    y = pl.pallas_call(
        _expert_body,
        grid_spec=pltpu.PrefetchScalarGridSpec(
            num_scalar_prefetch=1,
            grid=(NT,),
            in_specs=[
                pl.BlockSpec((TD, OBS), lambda i, e: (i, 0)),
                pl.BlockSpec((TD, 1), lambda i, e: (i, 0)),
                pl.BlockSpec((1, OBS, EH), lambda i, e: (e[i], 0, 0)),
                pl.BlockSpec((1, 1, EH), lambda i, e: (e[i], 0, 0)),
                pl.BlockSpec((1, EH, EH), lambda i, e: (e[i], 0, 0)),
                pl.BlockSpec((1, 1, EH), lambda i, e: (e[i], 0, 0)),
                pl.BlockSpec((1, EH, J), lambda i, e: (e[i], 0, 0)),
                pl.BlockSpec((1, 1, J), lambda i, e: (e[i], 0, 0)),
            ],
            out_specs=[pl.BlockSpec((TD, YP), lambda i, e: (i, 0))],
        ),
        out_shape=[jax.ShapeDtypeStruct((S, YP), _F32)],
        compiler_params=pltpu.CompilerParams(
            dimension_semantics=("arbitrary",)),
    )(eids, x_d, w_slot[:, None], W1f, c['ex_b1'][:, None, :],
      c['ex_W2'], c['ex_b2'][:, None, :], c['ex_W3'], c['ex_b3'][:, None, :])[0]

    mu_mix = (jnp.take(y, slot1, axis=0) + jnp.take(y, slot2, axis=0))[:, :J]

    TF = 512
    feats, log_std, log_sig = pl.pallas_call(
        _fn_body,
        grid=(B // TF,),
        in_specs=[
            pl.BlockSpec((TF, OBS), lambda i: (i, 0)),
            pl.BlockSpec((TF, LAT), lambda i: (i, 0)),
            pl.BlockSpec((TF, 1), lambda i: (i, 0)),
            _full_spec((OBS + LAT, HID)), _full_spec((1, HID)),
            _full_spec((1, HID)), _full_spec((1, HID)),
            _full_spec((HID, HID)), _full_spec((1, HID)),
            _full_spec((1, HID)), _full_spec((1, HID)),
            _full_spec((HID, HID)), _full_spec((1, HID)),
            _full_spec((HID, J)), _full_spec((1, J)),
            _full_spec((HID, POS)), _full_spec((1, POS)),
        ],
        out_specs=[
            pl.BlockSpec((TF, HID), lambda i: (i, 0)),
            pl.BlockSpec((TF, J), lambda i: (i, 0)),
            pl.BlockSpec((TF, POS), lambda i: (i, 0)),
        ],
        out_shape=[
            jax.ShapeDtypeStruct((B, HID), _F32),
            jax.ShapeDtypeStruct((B, J), _F32),
            jax.ShapeDtypeStruct((B, POS), _F32),
        ],
        compiler_params=pltpu.CompilerParams(
            dimension_semantics=("arbitrary",)),
    )(obs_t, z, mask_t,
      p['fn_W1'], r2(p['fn_b1']), r2(p['fn_g1']), r2(p['fn_be1']),
      p['fn_W2'], r2(p['fn_b2']), r2(p['fn_g2']), r2(p['fn_be2']),
      p['ls_W1'], r2(p['ls_b1']), p['ls_W2'], r2(p['ls_b2']),
      p['vh_W'], r2(p['vh_b']))

    return (mu_mix, mu_mix, log_std, log_sig, feats)


# speedup vs baseline: 1.6691x; 1.6691x over previous
import functools
import math

import jax
import jax.numpy as jnp
from jax import lax
from jax.experimental import pallas as pl
from jax.experimental.pallas import tpu as pltpu
from jax.experimental.pallas import tpu_sc as plsc

B = 4096
LAT = 64
OBS = 72
HID = 1024
J = 29
NCMD = 16
K = 8
TOPK = 2
GH = 256
POS = 75
EH = 512
LOG_STD_MIN = math.log(1e-4)
LOG_STD_MAX = math.log(5.0)

TD = 256
S = TOPK * B + K * TD
NT = S // TD
YP = 128
XP = 128

NSC = 2
NTILE = 16
NW = NSC * NTILE
TPW = B // NW
SPW = S // NW
SPT = S // NTILE

_F32 = jnp.float32


def _elu(x):
    return jnp.where(x > 0, x, jnp.exp(jnp.minimum(x, 0.0)) - 1.0)


def _softmax(x):
    m = jnp.max(x, axis=-1, keepdims=True)
    e = jnp.exp(x - m)
    return e / jnp.sum(e, axis=-1, keepdims=True)


def _ln(x, g, b):
    m = x.mean(-1, keepdims=True)
    v = ((x - m) ** 2).mean(-1, keepdims=True)
    return (x - m) * jax.lax.rsqrt(v + 1e-5) * g + b


def _gate_body(obs_ref, z_ref, chW1, chb1, chW2, chb2, gW1, gb1, gW2, gb2,
               gW3, gb3, lows, highs,
               xrow_out, wn_out, msk_out, rank_out, cnt_out, cnt_acc):
    i = pl.program_id(0)

    @pl.when(i == 0)
    def _():
        cnt_acc[...] = jnp.zeros_like(cnt_acc)

    obs = obs_ref[...]
    z = z_ref[...]
    oz = jnp.concatenate([obs, z], axis=-1)
    h = _elu(jnp.dot(oz, chW1[...], preferred_element_type=_F32) + chb1[...])
    cmd01 = jax.nn.sigmoid(jnp.dot(h, chW2[...], preferred_element_type=_F32) + chb2[...])
    lo = lows[...]
    cmd = lo + (highs[...] - lo) * cmd01
    g_in = jnp.concatenate([cmd, obs[:, NCMD:], z], axis=-1)
    g = _elu(jnp.dot(g_in, gW1[...], preferred_element_type=_F32) + gb1[...])
    g = _elu(jnp.dot(g, gW2[...], preferred_element_type=_F32) + gb2[...])
    logits = jnp.dot(g, gW3[...], preferred_element_type=_F32) + gb3[...]
    w = _softmax(logits)
    kidx = lax.broadcasted_iota(jnp.int32, w.shape, 1)
    m1 = jnp.max(w, axis=-1, keepdims=True)
    i1 = jnp.min(jnp.where(w == m1, kidx, K), axis=-1, keepdims=True)
    w2 = jnp.where(kidx == i1, -1.0, w)
    m2 = jnp.max(w2, axis=-1, keepdims=True)
    i2 = jnp.min(jnp.where(w2 == m2, kidx, K), axis=-1, keepdims=True)
    msk = (kidx == i1) | (kidx == i2)
    mskf = msk.astype(_F32)
    wm = jnp.where(msk, w, 0.0)
    wn = wm / jnp.sum(wm, axis=-1, keepdims=True)
    t = mskf.shape[0]
    r_i = lax.broadcasted_iota(jnp.int32, (t, t), 0)
    c_i = lax.broadcasted_iota(jnp.int32, (t, t), 1)
    tri = (r_i > c_i).astype(_F32)
    r_local = jnp.dot(tri, mskf, preferred_element_type=_F32)
    rank_out[...] = cnt_acc[...] + r_local
    cnt_acc[...] += jnp.sum(mskf, axis=0, keepdims=True)
    cnt_out[...] = cnt_acc[...]
    xrow_out[...] = jnp.concatenate(
        [g_in[:, :OBS], jnp.zeros((t, XP - OBS), _F32)], axis=-1)
    wn_out[...] = wn
    msk_out[...] = mskf


def _expert_body(eids_ref, xd_ref, ws_ref, W1f, b1f, W2, b2, W3, b3, y_ref):
    x = xd_ref[...]
    h = _elu(jnp.dot(x, W1f[0], preferred_element_type=_F32) + b1f[0])
    h = _elu(jnp.dot(h, W2[0], preferred_element_type=_F32) + b2[0])
    mu = jnp.dot(h, W3[0], preferred_element_type=_F32) + b3[0]
    wmu = ws_ref[...] * mu
    y_ref[...] = jnp.concatenate(
        [wmu, jnp.zeros((wmu.shape[0], YP - J), _F32)], axis=-1)


_CH = 64


def _sc_schedule_body(slot1_hbm, slot2_hbm, w1_hbm, w2_hbm, zi_hbm, zf_hbm,
                      ptok_hbm, pw_hbm,
                      tokloc, wloc, s1v, s2v, w1v, w2v,
                      rowt, roww, acct, accw, sh_tok, sh_w):
    cid = lax.axis_index("c")
    sid = lax.axis_index("s")
    wid = cid * NTILE + sid
    tbase = wid * TPW
    pltpu.sync_copy(zi_hbm, tokloc)
    pltpu.sync_copy(zf_hbm, wloc)
    pltpu.sync_copy(slot1_hbm.at[pl.ds(tbase, TPW)], s1v)
    pltpu.sync_copy(slot2_hbm.at[pl.ds(tbase, TPW)], s2v)
    pltpu.sync_copy(w1_hbm.at[pl.ds(tbase, TPW)], w1v)
    pltpu.sync_copy(w2_hbm.at[pl.ds(tbase, TPW)], w2v)
    for j in range(TPW // 16):
        tok = lax.iota(jnp.int32, 16) + (tbase + j * 16)
        idx1 = s1v[pl.ds(j * 16, 16)]
        plsc.store_scatter(tokloc, [idx1], tok)
        plsc.store_scatter(wloc, [idx1], w1v[pl.ds(j * 16, 16)])
        idx2 = s2v[pl.ds(j * 16, 16)]
        plsc.store_scatter(tokloc, [idx2], tok)
        plsc.store_scatter(wloc, [idx2], w2v[pl.ds(j * 16, 16)])
    pltpu.sync_copy(tokloc, sh_tok.at[sid])
    pltpu.sync_copy(wloc, sh_w.at[sid])
    plsc.subcore_barrier()
    strip = sid * SPT
    pltpu.sync_copy(sh_tok.at[0, pl.ds(strip, SPT)], acct)
    pltpu.sync_copy(sh_w.at[0, pl.ds(strip, SPT)], accw)

    def _row(r, carry):
        pltpu.sync_copy(sh_tok.at[r, pl.ds(strip, SPT)], rowt)
        pltpu.sync_copy(sh_w.at[r, pl.ds(strip, SPT)], roww)

        def _vec(i, c2):
            sl = pl.ds(i * 16, 16)
            acct[sl] = acct[sl] + rowt[sl]
            accw[sl] = accw[sl] + roww[sl]
            return c2

        return lax.fori_loop(0, SPT // 16, _vec, carry)

    lax.fori_loop(1, NTILE, _row, 0)
    pltpu.sync_copy(acct, ptok_hbm.at[pl.ds(cid * S + strip, SPT)])
    pltpu.sync_copy(accw, pw_hbm.at[pl.ds(cid * S + strip, SPT)])


def _sc_dispatch_body(ptok_hbm, pw_hbm, xrow_hbm, xd_hbm, ws_hbm,
                      tokA, tokB, wA, wB, rows, sem):
    cid = lax.axis_index("c")
    sid = lax.axis_index("s")
    wid = cid * NTILE + sid
    base = wid * SPW
    pltpu.sync_copy(ptok_hbm.at[pl.ds(base, SPW)], tokA)
    pltpu.sync_copy(ptok_hbm.at[pl.ds(S + base, SPW)], tokB)
    pltpu.sync_copy(pw_hbm.at[pl.ds(base, SPW)], wA)
    pltpu.sync_copy(pw_hbm.at[pl.ds(S + base, SPW)], wB)
    for i in range(SPW // 16):
        sl = pl.ds(i * 16, 16)
        tokA[sl] = tokA[sl] + tokB[sl]
        wA[sl] = wA[sl] + wB[sl]
    for g in range(SPW // _CH):
        pltpu.async_copy(
            xrow_hbm.at[tokA.at[pl.ds(g * _CH, _CH)]],
            rows.at[pl.ds(g * _CH, _CH), :], sem).wait()
    pltpu.sync_copy(rows, xd_hbm.at[pl.ds(base, SPW), :])
    pltpu.sync_copy(wA, ws_hbm.at[pl.ds(base, SPW)])


def _sc_combine_body(y_hbm, slot1_hbm, slot2_hbm, a_hbm, b_hbm,
                     s1v, s2v, rowsA, rowsB, sem):
    cid = lax.axis_index("c")
    sid = lax.axis_index("s")
    wid = cid * NTILE + sid
    base = wid * TPW
    pltpu.sync_copy(slot1_hbm.at[pl.ds(base, TPW)], s1v)
    pltpu.sync_copy(slot2_hbm.at[pl.ds(base, TPW)], s2v)
    for g in range(TPW // _CH):
        pltpu.async_copy(
            y_hbm.at[s1v.at[pl.ds(g * _CH, _CH)]],
            rowsA.at[pl.ds(g * _CH, _CH), :], sem).wait()
        pltpu.async_copy(
            y_hbm.at[s2v.at[pl.ds(g * _CH, _CH)]],
            rowsB.at[pl.ds(g * _CH, _CH), :], sem).wait()
    pltpu.sync_copy(rowsA, a_hbm.at[pl.ds(base, TPW), :])
    pltpu.sync_copy(rowsB, b_hbm.at[pl.ds(base, TPW), :])


def _fn_body(obs_ref, z_ref, mask_ref, ya_ref, yb_ref,
             fnW1, fnb1, g1, be1, fnW2, fnb2, g2,
             be2, lsW1, lsb1, lsW2, lsb2, vhW, vhb,
             mu_out, feats_out, ls_out, sig_out):
    mu_out[...] = (ya_ref[...] + yb_ref[...])[:, :J]
    oz = jnp.concatenate([obs_ref[...], z_ref[...]], axis=-1)
    x = jnp.dot(oz, fnW1[...], preferred_element_type=_F32) + fnb1[...]
    x = jax.nn.relu(_ln(x, g1[...], be1[...]))
    x = jnp.dot(x, fnW2[...], preferred_element_type=_F32) + fnb2[...]
    x = jax.nn.relu(_ln(x, g2[...], be2[...]))
    feats = x * mask_ref[...]
    h = jax.nn.relu(jnp.dot(feats, lsW1[...], preferred_element_type=_F32) + lsb1[...])
    ls = jnp.dot(h, lsW2[...], preferred_element_type=_F32) + lsb2[...]
    log_std = jnp.clip(ls, LOG_STD_MIN, LOG_STD_MAX)
    sr = jnp.dot(feats, vhW[...], preferred_element_type=_F32) + vhb[...]
    sigma = 0.05 + (0.5 - 0.05) * jax.nn.sigmoid(sr)
    feats_out[...] = feats
    ls_out[...] = log_std
    sig_out[...] = jnp.log(sigma)


def _row_spec(t, n):
    return pl.BlockSpec((t, n), lambda i: (0, 0) if t is None else (i, 0))


def _full_spec(shape):
    nd = len(shape)
    return pl.BlockSpec(shape, lambda i, _nd=nd: (0,) * _nd)


def kernel(z, obs_t, mask_t, params, consts):
    p, c = params, consts
    r2 = lambda a: a.reshape(1, -1)

    TG = 512
    xrow, wn, mskf, rank, cnt = pl.pallas_call(
        _gate_body,
        grid=(B // TG,),
        in_specs=[
            pl.BlockSpec((TG, OBS), lambda i: (i, 0)),
            pl.BlockSpec((TG, LAT), lambda i: (i, 0)),
            _full_spec((OBS + LAT, GH)), _full_spec((1, GH)),
            _full_spec((GH, NCMD)), _full_spec((1, NCMD)),
            _full_spec((OBS + LAT, GH)), _full_spec((1, GH)),
            _full_spec((GH, GH)), _full_spec((1, GH)),
            _full_spec((GH, K)), _full_spec((1, K)),
            _full_spec((1, NCMD)), _full_spec((1, NCMD)),
        ],
        out_specs=[
            pl.BlockSpec((TG, XP), lambda i: (i, 0)),
            pl.BlockSpec((TG, K), lambda i: (i, 0)),
            pl.BlockSpec((TG, K), lambda i: (i, 0)),
            pl.BlockSpec((TG, K), lambda i: (i, 0)),
            pl.BlockSpec((1, K), lambda i: (0, 0)),
        ],
        out_shape=[
            jax.ShapeDtypeStruct((B, XP), _F32),
            jax.ShapeDtypeStruct((B, K), _F32),
            jax.ShapeDtypeStruct((B, K), _F32),
            jax.ShapeDtypeStruct((B, K), _F32),
            jax.ShapeDtypeStruct((1, K), _F32),
        ],
        scratch_shapes=[pltpu.VMEM((1, K), _F32)],
        compiler_params=pltpu.CompilerParams(
            dimension_semantics=("arbitrary",)),
    )(obs_t, z, p['ch_W1'], r2(p['ch_b1']), p['ch_W2'], r2(p['ch_b2']),
      p['g_W1'], r2(p['g_b1']), p['g_W2'], r2(p['g_b2']),
      p['g_W3'], r2(p['g_b3']), r2(c['cmd_lows']), r2(c['cmd_highs']))

    n_k = cnt[0].astype(jnp.int32)
    pad_k = ((n_k + TD - 1) // TD) * TD
    cum_pad = jnp.cumsum(pad_k)
    start = cum_pad - pad_k
    slot_all = start[None, :] + rank.astype(jnp.int32)
    mskb = mskf > 0.5
    slot_m = jnp.where(mskb, slot_all, S)
    slot1 = jnp.min(slot_m, axis=-1)
    slot2 = jnp.max(jnp.where(mskb, slot_all, -1), axis=-1)
    w1 = jnp.sum(jnp.where(slot_m == slot1[:, None], wn, 0.0), axis=-1)
    w2 = jnp.sum(jnp.where(slot_m == slot2[:, None], wn, 0.0), axis=-1)
    tile_base = jnp.arange(NT, dtype=jnp.int32) * TD
    eids = jnp.minimum(
        jnp.sum((tile_base[:, None] >= cum_pad[None, :]).astype(jnp.int32),
                axis=-1), K - 1)

    mesh = plsc.VectorSubcoreMesh(core_axis_name="c", subcore_axis_name="s")
    ptok, pw = pl.kernel(
        _sc_schedule_body, mesh=mesh,
        out_type=[jax.ShapeDtypeStruct((NSC * S,), jnp.int32),
                  jax.ShapeDtypeStruct((NSC * S,), _F32)],
        scratch_types=[
            pltpu.VMEM((S,), jnp.int32), pltpu.VMEM((S,), _F32),
            pltpu.VMEM((TPW,), jnp.int32), pltpu.VMEM((TPW,), jnp.int32),
            pltpu.VMEM((TPW,), _F32), pltpu.VMEM((TPW,), _F32),
            pltpu.VMEM((SPT,), jnp.int32), pltpu.VMEM((SPT,), _F32),
            pltpu.VMEM((SPT,), jnp.int32), pltpu.VMEM((SPT,), _F32),
            pltpu.VMEM_SHARED((NTILE, S), jnp.int32),
            pltpu.VMEM_SHARED((NTILE, S), _F32),
        ],
        compiler_params=pltpu.CompilerParams(needs_layout_passes=False),
    )(slot1, slot2, w1, w2,
      jnp.zeros((S,), jnp.int32), jnp.zeros((S,), _F32))

    x_d, w_slot = pl.kernel(
        _sc_dispatch_body, mesh=mesh,
        out_type=[jax.ShapeDtypeStruct((S, XP), _F32),
                  jax.ShapeDtypeStruct((S,), _F32)],
        scratch_types=[
            pltpu.VMEM((SPW,), jnp.int32), pltpu.VMEM((SPW,), jnp.int32),
            pltpu.VMEM((SPW,), _F32), pltpu.VMEM((SPW,), _F32),
            pltpu.VMEM((SPW, XP), _F32),
            pltpu.SemaphoreType.DMA,
        ],
        compiler_params=pltpu.CompilerParams(needs_layout_passes=False),
    )(ptok, pw, xrow)

    W1f = (c['ex_W1'] * jnp.concatenate(
        [c['cmd_masks'][:, :, None],
         jnp.ones((K, OBS - NCMD, 1), _F32)], axis=1))
    W1f = jnp.concatenate([W1f, jnp.zeros((K, XP - OBS, EH), _F32)], axis=1)
    y = pl.pallas_call(
        _expert_body,
        grid_spec=pltpu.PrefetchScalarGridSpec(
            num_scalar_prefetch=1,
            grid=(NT,),
            in_specs=[
                pl.BlockSpec((TD, XP), lambda i, e: (i, 0)),
                pl.BlockSpec((TD, 1), lambda i, e: (i, 0)),
                pl.BlockSpec((1, XP, EH), lambda i, e: (e[i], 0, 0)),
                pl.BlockSpec((1, 1, EH), lambda i, e: (e[i], 0, 0)),
                pl.BlockSpec((1, EH, EH), lambda i, e: (e[i], 0, 0)),
                pl.BlockSpec((1, 1, EH), lambda i, e: (e[i], 0, 0)),
                pl.BlockSpec((1, EH, J), lambda i, e: (e[i], 0, 0)),
                pl.BlockSpec((1, 1, J), lambda i, e: (e[i], 0, 0)),
            ],
            out_specs=[pl.BlockSpec((TD, YP), lambda i, e: (i, 0))],
        ),
        out_shape=[jax.ShapeDtypeStruct((S, YP), _F32)],
        compiler_params=pltpu.CompilerParams(
            dimension_semantics=("arbitrary",)),
    )(eids, x_d, w_slot[:, None], W1f, c['ex_b1'][:, None, :],
      c['ex_W2'], c['ex_b2'][:, None, :], c['ex_W3'], c['ex_b3'][:, None, :])[0]

    ya, yb = pl.kernel(
        _sc_combine_body, mesh=mesh,
        out_type=[jax.ShapeDtypeStruct((B, YP), _F32),
                  jax.ShapeDtypeStruct((B, YP), _F32)],
        scratch_types=[
            pltpu.VMEM((TPW,), jnp.int32), pltpu.VMEM((TPW,), jnp.int32),
            pltpu.VMEM((TPW, YP), _F32), pltpu.VMEM((TPW, YP), _F32),
            pltpu.SemaphoreType.DMA,
        ],
        compiler_params=pltpu.CompilerParams(needs_layout_passes=False),
    )(y, slot1, slot2)

    TF = 512
    mu_mix, feats, log_std, log_sig = pl.pallas_call(
        _fn_body,
        grid=(B // TF,),
        in_specs=[
            pl.BlockSpec((TF, OBS), lambda i: (i, 0)),
            pl.BlockSpec((TF, LAT), lambda i: (i, 0)),
            pl.BlockSpec((TF, 1), lambda i: (i, 0)),
            pl.BlockSpec((TF, YP), lambda i: (i, 0)),
            pl.BlockSpec((TF, YP), lambda i: (i, 0)),
            _full_spec((OBS + LAT, HID)), _full_spec((1, HID)),
            _full_spec((1, HID)), _full_spec((1, HID)),
            _full_spec((HID, HID)), _full_spec((1, HID)),
            _full_spec((1, HID)), _full_spec((1, HID)),
            _full_spec((HID, HID)), _full_spec((1, HID)),
            _full_spec((HID, J)), _full_spec((1, J)),
            _full_spec((HID, POS)), _full_spec((1, POS)),
        ],
        out_specs=[
            pl.BlockSpec((TF, J), lambda i: (i, 0)),
            pl.BlockSpec((TF, HID), lambda i: (i, 0)),
            pl.BlockSpec((TF, J), lambda i: (i, 0)),
            pl.BlockSpec((TF, POS), lambda i: (i, 0)),
        ],
        out_shape=[
            jax.ShapeDtypeStruct((B, J), _F32),
            jax.ShapeDtypeStruct((B, HID), _F32),
            jax.ShapeDtypeStruct((B, J), _F32),
            jax.ShapeDtypeStruct((B, POS), _F32),
        ],
        compiler_params=pltpu.CompilerParams(
            dimension_semantics=("arbitrary",)),
    )(obs_t, z, mask_t, ya, yb,
      p['fn_W1'], r2(p['fn_b1']), r2(p['fn_g1']), r2(p['fn_be1']),
      p['fn_W2'], r2(p['fn_b2']), r2(p['fn_g2']), r2(p['fn_be2']),
      p['ls_W1'], r2(p['ls_b1']), p['ls_W2'], r2(p['ls_b2']),
      p['vh_W'], r2(p['vh_b']))

    return (mu_mix, mu_mix, log_std, log_sig, feats)


# speedup vs baseline: 1.7152x; 1.0276x over previous
import functools
import math

import jax
import jax.numpy as jnp
from jax import lax
from jax.experimental import pallas as pl
from jax.experimental.pallas import tpu as pltpu
from jax.experimental.pallas import tpu_sc as plsc

B = 4096
LAT = 64
OBS = 72
HID = 1024
J = 29
NCMD = 16
K = 8
TOPK = 2
GH = 256
POS = 75
EH = 512
LOG_STD_MIN = math.log(1e-4)
LOG_STD_MAX = math.log(5.0)

TD = 256
S = TOPK * B + K * TD
NT = S // TD
YP = 128
XP = 128

NSC = 2
NTILE = 16
NW = NSC * NTILE
TPW = B // NW
SPW = S // NW
SPT = S // NTILE

_F32 = jnp.float32


def _elu(x):
    return jnp.where(x > 0, x, jnp.exp(jnp.minimum(x, 0.0)) - 1.0)


def _softmax(x):
    m = jnp.max(x, axis=-1, keepdims=True)
    e = jnp.exp(x - m)
    return e / jnp.sum(e, axis=-1, keepdims=True)


def _ln(x, g, b):
    m = x.mean(-1, keepdims=True)
    v = ((x - m) ** 2).mean(-1, keepdims=True)
    return (x - m) * jax.lax.rsqrt(v + 1e-5) * g + b


def _gate_body(obs_ref, z_ref, chW1, chb1, chW2, chb2, gW1, gb1, gW2, gb2,
               gW3, gb3, lows, highs,
               xrow_out, wn_out, msk_out, rank_out, cnt_out, cnt_acc):
    i = pl.program_id(0)

    @pl.when(i == 0)
    def _():
        cnt_acc[...] = jnp.zeros_like(cnt_acc)

    obs = obs_ref[...]
    z = z_ref[...]
    oz = jnp.concatenate([obs, z], axis=-1)
    h = _elu(jnp.dot(oz, chW1[...], preferred_element_type=_F32) + chb1[...])
    cmd01 = jax.nn.sigmoid(jnp.dot(h, chW2[...], preferred_element_type=_F32) + chb2[...])
    lo = lows[...]
    cmd = lo + (highs[...] - lo) * cmd01
    g_in = jnp.concatenate([cmd, obs[:, NCMD:], z], axis=-1)
    g = _elu(jnp.dot(g_in, gW1[...], preferred_element_type=_F32) + gb1[...])
    g = _elu(jnp.dot(g, gW2[...], preferred_element_type=_F32) + gb2[...])
    logits = jnp.dot(g, gW3[...], preferred_element_type=_F32) + gb3[...]
    w = _softmax(logits)
    kidx = lax.broadcasted_iota(jnp.int32, w.shape, 1)
    m1 = jnp.max(w, axis=-1, keepdims=True)
    i1 = jnp.min(jnp.where(w == m1, kidx, K), axis=-1, keepdims=True)
    w2 = jnp.where(kidx == i1, -1.0, w)
    m2 = jnp.max(w2, axis=-1, keepdims=True)
    i2 = jnp.min(jnp.where(w2 == m2, kidx, K), axis=-1, keepdims=True)
    msk = (kidx == i1) | (kidx == i2)
    mskf = msk.astype(_F32)
    wm = jnp.where(msk, w, 0.0)
    wn = wm / jnp.sum(wm, axis=-1, keepdims=True)
    t = mskf.shape[0]
    r_i = lax.broadcasted_iota(jnp.int32, (t, t), 0)
    c_i = lax.broadcasted_iota(jnp.int32, (t, t), 1)
    tri = (r_i > c_i).astype(_F32)
    r_local = jnp.dot(tri, mskf, preferred_element_type=_F32)
    rank_out[...] = cnt_acc[...] + r_local
    cnt_acc[...] += jnp.sum(mskf, axis=0, keepdims=True)
    cnt_out[...] = cnt_acc[...]
    xrow_out[...] = jnp.concatenate(
        [g_in[:, :OBS], jnp.zeros((t, XP - OBS), _F32)], axis=-1)
    wn_out[...] = wn
    msk_out[...] = mskf


def _expert_body(eids_ref, xd_ref, ws_ref, W1f, b1f, W2, b2, W3, b3, y_ref):
    x = xd_ref[...]
    h = _elu(jnp.dot(x, W1f[0], preferred_element_type=_F32) + b1f[0])
    h = _elu(jnp.dot(h, W2[0], preferred_element_type=_F32) + b2[0])
    mu = jnp.dot(h, W3[0], preferred_element_type=_F32) + b3[0]
    wmu = ws_ref[...] * mu
    y_ref[...] = jnp.concatenate(
        [wmu, jnp.zeros((wmu.shape[0], YP - J), _F32)], axis=-1)


_CH = 64


def _sc_sched_dispatch_body(slot1_hbm, slot2_hbm, w1_hbm, w2_hbm,
                            zi_hbm, zf_hbm, xrow_hbm,
                            xd_hbm, ws_hbm,
                            tokloc, wloc, s1v, s2v, w1v, w2v,
                            mt, mw, acct, accw, rows, sem,
                            sh_tok, sh_w):
    cid = lax.axis_index("c")
    sid = lax.axis_index("s")
    tbase = sid * (B // NTILE)
    pltpu.sync_copy(zi_hbm, tokloc)
    pltpu.sync_copy(zf_hbm, wloc)
    npt = B // NTILE
    pltpu.sync_copy(slot1_hbm.at[pl.ds(tbase, npt)], s1v)
    pltpu.sync_copy(slot2_hbm.at[pl.ds(tbase, npt)], s2v)
    pltpu.sync_copy(w1_hbm.at[pl.ds(tbase, npt)], w1v)
    pltpu.sync_copy(w2_hbm.at[pl.ds(tbase, npt)], w2v)
    for j in range(npt // 16):
        tok = lax.iota(jnp.int32, 16) + (tbase + j * 16)
        idx1 = s1v[pl.ds(j * 16, 16)]
        plsc.store_scatter(tokloc, [idx1], tok)
        plsc.store_scatter(wloc, [idx1], w1v[pl.ds(j * 16, 16)])
        idx2 = s2v[pl.ds(j * 16, 16)]
        plsc.store_scatter(tokloc, [idx2], tok)
        plsc.store_scatter(wloc, [idx2], w2v[pl.ds(j * 16, 16)])
    pltpu.sync_copy(tokloc, sh_tok.at[sid])
    pltpu.sync_copy(wloc, sh_w.at[sid])
    plsc.subcore_barrier()
    strip = sid * SPT
    pltpu.sync_copy(sh_tok.at[:, pl.ds(strip, SPT)], mt)
    pltpu.sync_copy(sh_w.at[:, pl.ds(strip, SPT)], mw)
    for i in range(SPT // 16):
        sl = pl.ds(i * 16, 16)
        vt = mt[0, sl]
        vw = mw[0, sl]
        for r in range(1, NTILE):
            vt = vt + mt[r, sl]
            vw = vw + mw[r, sl]
        acct[sl] = vt
        accw[sl] = vw
    half = cid * (SPT // NSC)
    pltpu.sync_copy(accw.at[pl.ds(half, SPT // NSC)],
                    ws_hbm.at[pl.ds(strip + half, SPT // NSC)])
    for g in range(SPT // NSC // _CH):
        off = half + g * _CH
        pltpu.async_copy(
            xrow_hbm.at[acct.at[pl.ds(off, _CH)]],
            rows.at[pl.ds(g * _CH, _CH), :], sem).wait()
    pltpu.sync_copy(rows, xd_hbm.at[pl.ds(strip + half, SPT // NSC), :])


def _sc_combine_body(y_hbm, slot1_hbm, slot2_hbm, a_hbm, b_hbm,
                     s1v, s2v, rowsA, rowsB, sem):
    cid = lax.axis_index("c")
    sid = lax.axis_index("s")
    wid = cid * NTILE + sid
    base = wid * TPW
    pltpu.sync_copy(slot1_hbm.at[pl.ds(base, TPW)], s1v)
    pltpu.sync_copy(slot2_hbm.at[pl.ds(base, TPW)], s2v)
    for g in range(TPW // _CH):
        pltpu.async_copy(
            y_hbm.at[s1v.at[pl.ds(g * _CH, _CH)]],
            rowsA.at[pl.ds(g * _CH, _CH), :], sem).wait()
        pltpu.async_copy(
            y_hbm.at[s2v.at[pl.ds(g * _CH, _CH)]],
            rowsB.at[pl.ds(g * _CH, _CH), :], sem).wait()
    pltpu.sync_copy(rowsA, a_hbm.at[pl.ds(base, TPW), :])
    pltpu.sync_copy(rowsB, b_hbm.at[pl.ds(base, TPW), :])


def _fn_body(obs_ref, z_ref, mask_ref, ya_ref, yb_ref,
             fnW1, fnb1, g1, be1, fnW2, fnb2, g2,
             be2, lsW1, lsb1, lsW2, lsb2, vhW, vhb,
             mu_out, feats_out, ls_out, sig_out):
    mu_out[...] = (ya_ref[...] + yb_ref[...])[:, :J]
    oz = jnp.concatenate([obs_ref[...], z_ref[...]], axis=-1)
    x = jnp.dot(oz, fnW1[...], preferred_element_type=_F32) + fnb1[...]
    x = jax.nn.relu(_ln(x, g1[...], be1[...]))
    x = jnp.dot(x, fnW2[...], preferred_element_type=_F32) + fnb2[...]
    x = jax.nn.relu(_ln(x, g2[...], be2[...]))
    feats = x * mask_ref[...]
    h = jax.nn.relu(jnp.dot(feats, lsW1[...], preferred_element_type=_F32) + lsb1[...])
    ls = jnp.dot(h, lsW2[...], preferred_element_type=_F32) + lsb2[...]
    log_std = jnp.clip(ls, LOG_STD_MIN, LOG_STD_MAX)
    sr = jnp.dot(feats, vhW[...], preferred_element_type=_F32) + vhb[...]
    sigma = 0.05 + (0.5 - 0.05) * jax.nn.sigmoid(sr)
    feats_out[...] = feats
    ls_out[...] = log_std
    sig_out[...] = jnp.log(sigma)


def _row_spec(t, n):
    return pl.BlockSpec((t, n), lambda i: (0, 0) if t is None else (i, 0))


def _full_spec(shape):
    nd = len(shape)
    return pl.BlockSpec(shape, lambda i, _nd=nd: (0,) * _nd)


def kernel(z, obs_t, mask_t, params, consts):
    p, c = params, consts
    r2 = lambda a: a.reshape(1, -1)

    TG = 512
    xrow, wn, mskf, rank, cnt = pl.pallas_call(
        _gate_body,
        grid=(B // TG,),
        in_specs=[
            pl.BlockSpec((TG, OBS), lambda i: (i, 0)),
            pl.BlockSpec((TG, LAT), lambda i: (i, 0)),
            _full_spec((OBS + LAT, GH)), _full_spec((1, GH)),
            _full_spec((GH, NCMD)), _full_spec((1, NCMD)),
            _full_spec((OBS + LAT, GH)), _full_spec((1, GH)),
            _full_spec((GH, GH)), _full_spec((1, GH)),
            _full_spec((GH, K)), _full_spec((1, K)),
            _full_spec((1, NCMD)), _full_spec((1, NCMD)),
        ],
        out_specs=[
            pl.BlockSpec((TG, XP), lambda i: (i, 0)),
            pl.BlockSpec((TG, K), lambda i: (i, 0)),
            pl.BlockSpec((TG, K), lambda i: (i, 0)),
            pl.BlockSpec((TG, K), lambda i: (i, 0)),
            pl.BlockSpec((1, K), lambda i: (0, 0)),
        ],
        out_shape=[
            jax.ShapeDtypeStruct((B, XP), _F32),
            jax.ShapeDtypeStruct((B, K), _F32),
            jax.ShapeDtypeStruct((B, K), _F32),
            jax.ShapeDtypeStruct((B, K), _F32),
            jax.ShapeDtypeStruct((1, K), _F32),
        ],
        scratch_shapes=[pltpu.VMEM((1, K), _F32)],
        compiler_params=pltpu.CompilerParams(
            dimension_semantics=("arbitrary",)),
    )(obs_t, z, p['ch_W1'], r2(p['ch_b1']), p['ch_W2'], r2(p['ch_b2']),
      p['g_W1'], r2(p['g_b1']), p['g_W2'], r2(p['g_b2']),
      p['g_W3'], r2(p['g_b3']), r2(c['cmd_lows']), r2(c['cmd_highs']))

    n_k = cnt[0].astype(jnp.int32)
    pad_k = ((n_k + TD - 1) // TD) * TD
    cum_pad = jnp.cumsum(pad_k)
    start = cum_pad - pad_k
    slot_all = start[None, :] + rank.astype(jnp.int32)
    mskb = mskf > 0.5
    slot_m = jnp.where(mskb, slot_all, S)
    slot1 = jnp.min(slot_m, axis=-1)
    slot2 = jnp.max(jnp.where(mskb, slot_all, -1), axis=-1)
    w1 = jnp.sum(jnp.where(slot_m == slot1[:, None], wn, 0.0), axis=-1)
    w2 = jnp.sum(jnp.where(slot_m == slot2[:, None], wn, 0.0), axis=-1)
    tile_base = jnp.arange(NT, dtype=jnp.int32) * TD
    eids = jnp.minimum(
        jnp.sum((tile_base[:, None] >= cum_pad[None, :]).astype(jnp.int32),
                axis=-1), K - 1)

    mesh = plsc.VectorSubcoreMesh(core_axis_name="c", subcore_axis_name="s")
    x_d, w_slot = pl.kernel(
        _sc_sched_dispatch_body, mesh=mesh,
        out_type=[jax.ShapeDtypeStruct((S, XP), _F32),
                  jax.ShapeDtypeStruct((S,), _F32)],
        scratch_types=[
            pltpu.VMEM((S,), jnp.int32), pltpu.VMEM((S,), _F32),
            pltpu.VMEM((B // NTILE,), jnp.int32),
            pltpu.VMEM((B // NTILE,), jnp.int32),
            pltpu.VMEM((B // NTILE,), _F32), pltpu.VMEM((B // NTILE,), _F32),
            pltpu.VMEM((NTILE, SPT), jnp.int32), pltpu.VMEM((NTILE, SPT), _F32),
            pltpu.VMEM((SPT,), jnp.int32), pltpu.VMEM((SPT,), _F32),
            pltpu.VMEM((SPT // NSC, XP), _F32),
            pltpu.SemaphoreType.DMA,
            pltpu.VMEM_SHARED((NTILE, S), jnp.int32),
            pltpu.VMEM_SHARED((NTILE, S), _F32),
        ],
        compiler_params=pltpu.CompilerParams(needs_layout_passes=False),
    )(slot1, slot2, w1, w2,
      jnp.zeros((S,), jnp.int32), jnp.zeros((S,), _F32), xrow)

    W1f = (c['ex_W1'] * jnp.concatenate(
        [c['cmd_masks'][:, :, None],
         jnp.ones((K, OBS - NCMD, 1), _F32)], axis=1))
    W1f = jnp.concatenate([W1f, jnp.zeros((K, XP - OBS, EH), _F32)], axis=1)
    y = pl.pallas_call(
        _expert_body,
        grid_spec=pltpu.PrefetchScalarGridSpec(
            num_scalar_prefetch=1,
            grid=(NT,),
            in_specs=[
                pl.BlockSpec((TD, XP), lambda i, e: (i, 0)),
                pl.BlockSpec((TD, 1), lambda i, e: (i, 0)),
                pl.BlockSpec((1, XP, EH), lambda i, e: (e[i], 0, 0)),
                pl.BlockSpec((1, 1, EH), lambda i, e: (e[i], 0, 0)),
                pl.BlockSpec((1, EH, EH), lambda i, e: (e[i], 0, 0)),
                pl.BlockSpec((1, 1, EH), lambda i, e: (e[i], 0, 0)),
                pl.BlockSpec((1, EH, J), lambda i, e: (e[i], 0, 0)),
                pl.BlockSpec((1, 1, J), lambda i, e: (e[i], 0, 0)),
            ],
            out_specs=[pl.BlockSpec((TD, YP), lambda i, e: (i, 0))],
        ),
        out_shape=[jax.ShapeDtypeStruct((S, YP), _F32)],
        compiler_params=pltpu.CompilerParams(
            dimension_semantics=("arbitrary",)),
    )(eids, x_d, w_slot[:, None], W1f, c['ex_b1'][:, None, :],
      c['ex_W2'], c['ex_b2'][:, None, :], c['ex_W3'], c['ex_b3'][:, None, :])[0]

    ya, yb = pl.kernel(
        _sc_combine_body, mesh=mesh,
        out_type=[jax.ShapeDtypeStruct((B, YP), _F32),
                  jax.ShapeDtypeStruct((B, YP), _F32)],
        scratch_types=[
            pltpu.VMEM((TPW,), jnp.int32), pltpu.VMEM((TPW,), jnp.int32),
            pltpu.VMEM((TPW, YP), _F32), pltpu.VMEM((TPW, YP), _F32),
            pltpu.SemaphoreType.DMA,
        ],
        compiler_params=pltpu.CompilerParams(needs_layout_passes=False),
    )(y, slot1, slot2)

    TF = 512
    mu_mix, feats, log_std, log_sig = pl.pallas_call(
        _fn_body,
        grid=(B // TF,),
        in_specs=[
            pl.BlockSpec((TF, OBS), lambda i: (i, 0)),
            pl.BlockSpec((TF, LAT), lambda i: (i, 0)),
            pl.BlockSpec((TF, 1), lambda i: (i, 0)),
            pl.BlockSpec((TF, YP), lambda i: (i, 0)),
            pl.BlockSpec((TF, YP), lambda i: (i, 0)),
            _full_spec((OBS + LAT, HID)), _full_spec((1, HID)),
            _full_spec((1, HID)), _full_spec((1, HID)),
            _full_spec((HID, HID)), _full_spec((1, HID)),
            _full_spec((1, HID)), _full_spec((1, HID)),
            _full_spec((HID, HID)), _full_spec((1, HID)),
            _full_spec((HID, J)), _full_spec((1, J)),
            _full_spec((HID, POS)), _full_spec((1, POS)),
        ],
        out_specs=[
            pl.BlockSpec((TF, J), lambda i: (i, 0)),
            pl.BlockSpec((TF, HID), lambda i: (i, 0)),
            pl.BlockSpec((TF, J), lambda i: (i, 0)),
            pl.BlockSpec((TF, POS), lambda i: (i, 0)),
        ],
        out_shape=[
            jax.ShapeDtypeStruct((B, J), _F32),
            jax.ShapeDtypeStruct((B, HID), _F32),
            jax.ShapeDtypeStruct((B, J), _F32),
            jax.ShapeDtypeStruct((B, POS), _F32),
        ],
        compiler_params=pltpu.CompilerParams(
            dimension_semantics=("arbitrary",)),
    )(obs_t, z, mask_t, ya, yb,
      p['fn_W1'], r2(p['fn_b1']), r2(p['fn_g1']), r2(p['fn_be1']),
      p['fn_W2'], r2(p['fn_b2']), r2(p['fn_g2']), r2(p['fn_be2']),
      p['ls_W1'], r2(p['ls_b1']), p['ls_W2'], r2(p['ls_b2']),
      p['vh_W'], r2(p['vh_b']))

    return (mu_mix, mu_mix, log_std, log_sig, feats)


# speedup vs baseline: 2.8813x; 1.6799x over previous
import functools
import math

import jax
import jax.numpy as jnp
from jax import lax
from jax.experimental import pallas as pl
from jax.experimental.pallas import tpu as pltpu
from jax.experimental.pallas import tpu_sc as plsc

B = 4096
LAT = 64
OBS = 72
HID = 1024
J = 29
NCMD = 16
K = 8
TOPK = 2
GH = 256
POS = 75
EH = 512
LOG_STD_MIN = math.log(1e-4)
LOG_STD_MAX = math.log(5.0)

TD = 256
S = TOPK * B + K * TD
NT = S // TD
YP = 128
XP = 128

NSC = 2
NTILE = 16
NW = NSC * NTILE
TPW = B // NW
SPW = S // NW
SPT = S // NTILE

_F32 = jnp.float32


def _elu(x):
    return jnp.where(x > 0, x, jnp.exp(jnp.minimum(x, 0.0)) - 1.0)


def _softmax(x):
    m = jnp.max(x, axis=-1, keepdims=True)
    e = jnp.exp(x - m)
    return e / jnp.sum(e, axis=-1, keepdims=True)


def _ln(x, g, b):
    m = x.mean(-1, keepdims=True)
    v = ((x - m) ** 2).mean(-1, keepdims=True)
    return (x - m) * jax.lax.rsqrt(v + 1e-5) * g + b


def _gate_body(obs_ref, z_ref, chW1, chb1, chW2, chb2, gW1, gb1, gW2, gb2,
               gW3, gb3, lows, highs,
               xrow_out, wn_out, msk_out, rank_out, cnt_out, cnt_acc):
    i = pl.program_id(0)

    @pl.when(i == 0)
    def _():
        cnt_acc[...] = jnp.zeros_like(cnt_acc)

    obs = obs_ref[...]
    z = z_ref[...]
    oz = jnp.concatenate([obs, z], axis=-1)
    h = _elu(jnp.dot(oz, chW1[...], preferred_element_type=_F32) + chb1[...])
    cmd01 = jax.nn.sigmoid(jnp.dot(h, chW2[...], preferred_element_type=_F32) + chb2[...])
    lo = lows[...]
    cmd = lo + (highs[...] - lo) * cmd01
    g_in = jnp.concatenate([cmd, obs[:, NCMD:], z], axis=-1)
    g = _elu(jnp.dot(g_in, gW1[...], preferred_element_type=_F32) + gb1[...])
    g = _elu(jnp.dot(g, gW2[...], preferred_element_type=_F32) + gb2[...])
    logits = jnp.dot(g, gW3[...], preferred_element_type=_F32) + gb3[...]
    w = _softmax(logits)
    kidx = lax.broadcasted_iota(jnp.int32, w.shape, 1)
    m1 = jnp.max(w, axis=-1, keepdims=True)
    i1 = jnp.min(jnp.where(w == m1, kidx, K), axis=-1, keepdims=True)
    w2 = jnp.where(kidx == i1, -1.0, w)
    m2 = jnp.max(w2, axis=-1, keepdims=True)
    i2 = jnp.min(jnp.where(w2 == m2, kidx, K), axis=-1, keepdims=True)
    msk = (kidx == i1) | (kidx == i2)
    mskf = msk.astype(_F32)
    wm = jnp.where(msk, w, 0.0)
    wn = wm / jnp.sum(wm, axis=-1, keepdims=True)
    t = mskf.shape[0]
    r_i = lax.broadcasted_iota(jnp.int32, (t, t), 0)
    c_i = lax.broadcasted_iota(jnp.int32, (t, t), 1)
    tri = (r_i > c_i).astype(_F32)
    r_local = jnp.dot(tri, mskf, preferred_element_type=_F32)
    rank_out[...] = cnt_acc[...] + r_local
    cnt_acc[...] += jnp.sum(mskf, axis=0, keepdims=True)
    cnt_out[...] = cnt_acc[...]
    xrow_out[...] = jnp.concatenate(
        [g_in[:, :OBS], jnp.zeros((t, XP - OBS), _F32)], axis=-1)
    wn_out[...] = wn
    msk_out[...] = mskf


def _expert_body(eids_ref, xd_ref, W1f, b1f, W2, b2, W3, b3, y_ref):
    x = xd_ref[...]
    h = _elu(jnp.dot(x, W1f[0], preferred_element_type=_F32) + b1f[0])
    h = _elu(jnp.dot(h, W2[0], preferred_element_type=_F32) + b2[0])
    mu = jnp.dot(h, W3[0], preferred_element_type=_F32) + b3[0]
    wmu = x[:, XP - 1:XP] * mu
    y_ref[...] = jnp.concatenate(
        [wmu, jnp.zeros((wmu.shape[0], YP - J), _F32)], axis=-1)


_CH = 64


def _sc_push_dispatch_body(slot1_hbm, slot2_hbm, w1_hbm, w2_hbm, xrow_hbm,
                           xd_hbm,
                           s1v, s2v, w1v, w2v, xbuf, sem):
    cid = lax.axis_index("c")
    sid = lax.axis_index("s")
    wid = cid * NTILE + sid
    tbase = wid * TPW
    pltpu.sync_copy(xrow_hbm.at[pl.ds(tbase, TPW), :], xbuf)
    pltpu.sync_copy(slot1_hbm.at[pl.ds(tbase, TPW)], s1v)
    pltpu.sync_copy(slot2_hbm.at[pl.ds(tbase, TPW)], s2v)
    pltpu.sync_copy(w1_hbm.at[pl.ds(tbase, TPW)], w1v)
    pltpu.sync_copy(w2_hbm.at[pl.ds(tbase, TPW)], w2v)
    lastcol = jnp.full((16,), XP - 1, jnp.int32)
    for j in range(TPW // 16):
        rows = lax.iota(jnp.int32, 16) + j * 16
        plsc.store_scatter(xbuf, [rows, lastcol], w1v[pl.ds(j * 16, 16)])
    pltpu.async_copy(xbuf, xd_hbm.at[s1v], sem).wait()
    for j in range(TPW // 16):
        rows = lax.iota(jnp.int32, 16) + j * 16
        plsc.store_scatter(xbuf, [rows, lastcol], w2v[pl.ds(j * 16, 16)])
    pltpu.async_copy(xbuf, xd_hbm.at[s2v], sem).wait()


def _sc_combine_body(y_hbm, slot1_hbm, slot2_hbm, a_hbm, b_hbm,
                     s1v, s2v, rowsA, rowsB, sem):
    cid = lax.axis_index("c")
    sid = lax.axis_index("s")
    wid = cid * NTILE + sid
    base = wid * TPW
    pltpu.sync_copy(slot1_hbm.at[pl.ds(base, TPW)], s1v)
    pltpu.sync_copy(slot2_hbm.at[pl.ds(base, TPW)], s2v)
    for g in range(TPW // _CH):
        pltpu.async_copy(
            y_hbm.at[s1v.at[pl.ds(g * _CH, _CH)]],
            rowsA.at[pl.ds(g * _CH, _CH), :], sem).wait()
        pltpu.async_copy(
            y_hbm.at[s2v.at[pl.ds(g * _CH, _CH)]],
            rowsB.at[pl.ds(g * _CH, _CH), :], sem).wait()
    pltpu.sync_copy(rowsA, a_hbm.at[pl.ds(base, TPW), :])
    pltpu.sync_copy(rowsB, b_hbm.at[pl.ds(base, TPW), :])


def _fn_body(obs_ref, z_ref, mask_ref, ya_ref, yb_ref,
             fnW1, fnb1, g1, be1, fnW2, fnb2, g2,
             be2, lsW1, lsb1, lsW2, lsb2, vhW, vhb,
             mu_out, feats_out, ls_out, sig_out):
    mu_out[...] = (ya_ref[...] + yb_ref[...])[:, :J]
    oz = jnp.concatenate([obs_ref[...], z_ref[...]], axis=-1)
    x = jnp.dot(oz, fnW1[...], preferred_element_type=_F32) + fnb1[...]
    x = jax.nn.relu(_ln(x, g1[...], be1[...]))
    x = jnp.dot(x, fnW2[...], preferred_element_type=_F32) + fnb2[...]
    x = jax.nn.relu(_ln(x, g2[...], be2[...]))
    feats = x * mask_ref[...]
    h = jax.nn.relu(jnp.dot(feats, lsW1[...], preferred_element_type=_F32) + lsb1[...])
    ls = jnp.dot(h, lsW2[...], preferred_element_type=_F32) + lsb2[...]
    log_std = jnp.clip(ls, LOG_STD_MIN, LOG_STD_MAX)
    sr = jnp.dot(feats, vhW[...], preferred_element_type=_F32) + vhb[...]
    sigma = 0.05 + (0.5 - 0.05) * jax.nn.sigmoid(sr)
    feats_out[...] = feats
    ls_out[...] = log_std
    sig_out[...] = jnp.log(sigma)


def _row_spec(t, n):
    return pl.BlockSpec((t, n), lambda i: (0, 0) if t is None else (i, 0))


def _full_spec(shape):
    nd = len(shape)
    return pl.BlockSpec(shape, lambda i, _nd=nd: (0,) * _nd)


def kernel(z, obs_t, mask_t, params, consts):
    p, c = params, consts
    r2 = lambda a: a.reshape(1, -1)

    TG = 512
    xrow, wn, mskf, rank, cnt = pl.pallas_call(
        _gate_body,
        grid=(B // TG,),
        in_specs=[
            pl.BlockSpec((TG, OBS), lambda i: (i, 0)),
            pl.BlockSpec((TG, LAT), lambda i: (i, 0)),
            _full_spec((OBS + LAT, GH)), _full_spec((1, GH)),
            _full_spec((GH, NCMD)), _full_spec((1, NCMD)),
            _full_spec((OBS + LAT, GH)), _full_spec((1, GH)),
            _full_spec((GH, GH)), _full_spec((1, GH)),
            _full_spec((GH, K)), _full_spec((1, K)),
            _full_spec((1, NCMD)), _full_spec((1, NCMD)),
        ],
        out_specs=[
            pl.BlockSpec((TG, XP), lambda i: (i, 0)),
            pl.BlockSpec((TG, K), lambda i: (i, 0)),
            pl.BlockSpec((TG, K), lambda i: (i, 0)),
            pl.BlockSpec((TG, K), lambda i: (i, 0)),
            pl.BlockSpec((1, K), lambda i: (0, 0)),
        ],
        out_shape=[
            jax.ShapeDtypeStruct((B, XP), _F32),
            jax.ShapeDtypeStruct((B, K), _F32),
            jax.ShapeDtypeStruct((B, K), _F32),
            jax.ShapeDtypeStruct((B, K), _F32),
            jax.ShapeDtypeStruct((1, K), _F32),
        ],
        scratch_shapes=[pltpu.VMEM((1, K), _F32)],
        compiler_params=pltpu.CompilerParams(
            dimension_semantics=("arbitrary",)),
    )(obs_t, z, p['ch_W1'], r2(p['ch_b1']), p['ch_W2'], r2(p['ch_b2']),
      p['g_W1'], r2(p['g_b1']), p['g_W2'], r2(p['g_b2']),
      p['g_W3'], r2(p['g_b3']), r2(c['cmd_lows']), r2(c['cmd_highs']))

    n_k = cnt[0].astype(jnp.int32)
    pad_k = ((n_k + TD - 1) // TD) * TD
    cum_pad = jnp.cumsum(pad_k)
    start = cum_pad - pad_k
    slot_all = start[None, :] + rank.astype(jnp.int32)
    mskb = mskf > 0.5
    slot_m = jnp.where(mskb, slot_all, S)
    slot1 = jnp.min(slot_m, axis=-1)
    slot2 = jnp.max(jnp.where(mskb, slot_all, -1), axis=-1)
    w1 = jnp.sum(jnp.where(slot_m == slot1[:, None], wn, 0.0), axis=-1)
    w2 = jnp.sum(jnp.where(slot_m == slot2[:, None], wn, 0.0), axis=-1)
    tile_base = jnp.arange(NT, dtype=jnp.int32) * TD
    eids = jnp.minimum(
        jnp.sum((tile_base[:, None] >= cum_pad[None, :]).astype(jnp.int32),
                axis=-1), K - 1)

    mesh = plsc.VectorSubcoreMesh(core_axis_name="c", subcore_axis_name="s")
    x_d = pl.kernel(
        _sc_push_dispatch_body, mesh=mesh,
        out_type=jax.ShapeDtypeStruct((S, XP), _F32),
        scratch_types=[
            pltpu.VMEM((TPW,), jnp.int32), pltpu.VMEM((TPW,), jnp.int32),
            pltpu.VMEM((TPW,), _F32), pltpu.VMEM((TPW,), _F32),
            pltpu.VMEM((TPW, XP), _F32),
            pltpu.SemaphoreType.DMA,
        ],
        compiler_params=pltpu.CompilerParams(needs_layout_passes=False),
    )(slot1, slot2, w1, w2, xrow)

    W1f = (c['ex_W1'] * jnp.concatenate(
        [c['cmd_masks'][:, :, None],
         jnp.ones((K, OBS - NCMD, 1), _F32)], axis=1))
    W1f = jnp.concatenate([W1f, jnp.zeros((K, XP - OBS, EH), _F32)], axis=1)
    y = pl.pallas_call(
        _expert_body,
        grid_spec=pltpu.PrefetchScalarGridSpec(
            num_scalar_prefetch=1,
            grid=(NT,),
            in_specs=[
                pl.BlockSpec((TD, XP), lambda i, e: (i, 0)),
                pl.BlockSpec((1, XP, EH), lambda i, e: (e[i], 0, 0)),
                pl.BlockSpec((1, 1, EH), lambda i, e: (e[i], 0, 0)),
                pl.BlockSpec((1, EH, EH), lambda i, e: (e[i], 0, 0)),
                pl.BlockSpec((1, 1, EH), lambda i, e: (e[i], 0, 0)),
                pl.BlockSpec((1, EH, J), lambda i, e: (e[i], 0, 0)),
                pl.BlockSpec((1, 1, J), lambda i, e: (e[i], 0, 0)),
            ],
            out_specs=[pl.BlockSpec((TD, YP), lambda i, e: (i, 0))],
        ),
        out_shape=[jax.ShapeDtypeStruct((S, YP), _F32)],
        compiler_params=pltpu.CompilerParams(
            dimension_semantics=("arbitrary",)),
    )(eids, x_d, W1f, c['ex_b1'][:, None, :],
      c['ex_W2'], c['ex_b2'][:, None, :], c['ex_W3'], c['ex_b3'][:, None, :])[0]

    ya, yb = pl.kernel(
        _sc_combine_body, mesh=mesh,
        out_type=[jax.ShapeDtypeStruct((B, YP), _F32),
                  jax.ShapeDtypeStruct((B, YP), _F32)],
        scratch_types=[
            pltpu.VMEM((TPW,), jnp.int32), pltpu.VMEM((TPW,), jnp.int32),
            pltpu.VMEM((TPW, YP), _F32), pltpu.VMEM((TPW, YP), _F32),
            pltpu.SemaphoreType.DMA,
        ],
        compiler_params=pltpu.CompilerParams(needs_layout_passes=False),
    )(y, slot1, slot2)

    TF = 512
    mu_mix, feats, log_std, log_sig = pl.pallas_call(
        _fn_body,
        grid=(B // TF,),
        in_specs=[
            pl.BlockSpec((TF, OBS), lambda i: (i, 0)),
            pl.BlockSpec((TF, LAT), lambda i: (i, 0)),
            pl.BlockSpec((TF, 1), lambda i: (i, 0)),
            pl.BlockSpec((TF, YP), lambda i: (i, 0)),
            pl.BlockSpec((TF, YP), lambda i: (i, 0)),
            _full_spec((OBS + LAT, HID)), _full_spec((1, HID)),
            _full_spec((1, HID)), _full_spec((1, HID)),
            _full_spec((HID, HID)), _full_spec((1, HID)),
            _full_spec((1, HID)), _full_spec((1, HID)),
            _full_spec((HID, HID)), _full_spec((1, HID)),
            _full_spec((HID, J)), _full_spec((1, J)),
            _full_spec((HID, POS)), _full_spec((1, POS)),
        ],
        out_specs=[
            pl.BlockSpec((TF, J), lambda i: (i, 0)),
            pl.BlockSpec((TF, HID), lambda i: (i, 0)),
            pl.BlockSpec((TF, J), lambda i: (i, 0)),
            pl.BlockSpec((TF, POS), lambda i: (i, 0)),
        ],
        out_shape=[
            jax.ShapeDtypeStruct((B, J), _F32),
            jax.ShapeDtypeStruct((B, HID), _F32),
            jax.ShapeDtypeStruct((B, J), _F32),
            jax.ShapeDtypeStruct((B, POS), _F32),
        ],
        compiler_params=pltpu.CompilerParams(
            dimension_semantics=("arbitrary",)),
    )(obs_t, z, mask_t, ya, yb,
      p['fn_W1'], r2(p['fn_b1']), r2(p['fn_g1']), r2(p['fn_be1']),
      p['fn_W2'], r2(p['fn_b2']), r2(p['fn_g2']), r2(p['fn_be2']),
      p['ls_W1'], r2(p['ls_b1']), p['ls_W2'], r2(p['ls_b2']),
      p['vh_W'], r2(p['vh_b']))

    return (mu_mix, mu_mix, log_std, log_sig, feats)
